# baseline (speedup 1.0000x reference)
import jax
import jax.numpy as jnp
from jax import lax
from jax.experimental import pallas as pl
from jax.experimental.pallas import tpu as pltpu
from jax.experimental.pallas import tpu_sc as plsc

_N_ENT = 100000
_N_REL = 1000
_D = 64

_ENT_W = 128
_ENT_R = (_N_ENT * _D) // _ENT_W
_BR = 2000


def _tc_add_body(a_ref, b_ref, o_ref):
    o_ref[...] = a_ref[...] + b_ref[...]


_tc_ent_add = pl.pallas_call(
    _tc_add_body,
    grid=(_ENT_R // _BR,),
    in_specs=[
        pl.BlockSpec((_BR, _ENT_W), lambda i: (i, 0)),
        pl.BlockSpec((_BR, _ENT_W), lambda i: (i, 0)),
    ],
    out_specs=pl.BlockSpec((_BR, _ENT_W), lambda i: (i, 0)),
    out_shape=jax.ShapeDtypeStruct((_ENT_R, _ENT_W), jnp.float32),
)

_NUM_CORES = 2
_NUM_SUBCORES = 16
_LANES = 16
_RCH = 32
_REL_LAST = _N_REL - _RCH


def _add_rows(a, b, rows, rpi):
    def step(i, _):
        r0 = i * rpi
        for r in range(rpi):
            for j in range(_D // _LANES):
                sl = pl.ds(j * _LANES, _LANES)
                a[r0 + r, sl] = a[r0 + r, sl] + b[r0 + r, sl]
        return 0

    lax.fori_loop(0, rows // rpi, step, 0)


def _sc_rel_body(r0, r1, out_r, a, b):
    wid = lax.axis_index("s") * _NUM_CORES + lax.axis_index("c")
    base = pl.multiple_of(jnp.minimum(wid * _RCH, _REL_LAST), 8)
    rows = pl.ds(base, _RCH)
    pltpu.sync_copy(r0.at[rows], a)
    pltpu.sync_copy(r1.at[rows], b)
    _add_rows(a, b, _RCH, 8)
    pltpu.sync_copy(a, out_r.at[rows])


_sc_rel_add = pl.kernel(
    _sc_rel_body,
    out_type=jax.ShapeDtypeStruct((_N_REL, _D), jnp.float32),
    mesh=plsc.VectorSubcoreMesh(
        core_axis_name="c",
        subcore_axis_name="s",
        num_cores=_NUM_CORES,
        num_subcores=_NUM_SUBCORES,
    ),
    scratch_types=[
        pltpu.VMEM((_RCH, _D), jnp.float32),
        pltpu.VMEM((_RCH, _D), jnp.float32),
    ],
)


def kernel(inputs, ent_embeds_0, rel_embeds_0, ent_embeds_1, rel_embeds_1):
    del inputs
    e0 = ent_embeds_0.reshape(_ENT_R, _ENT_W)
    e1 = ent_embeds_1.reshape(_ENT_R, _ENT_W)
    out_ent = _tc_ent_add(e0, e1).reshape(_N_ENT, _D)
    out_rel = _sc_rel_add(rel_embeds_0, rel_embeds_1)
    return (out_ent, out_rel)


# speedup vs baseline: 1.2677x; 1.2677x over previous
import jax
import jax.numpy as jnp
from jax import lax
from jax.experimental import pallas as pl
from jax.experimental.pallas import tpu as pltpu
from jax.experimental.pallas import tpu_sc as plsc

_N_ENT = 100000
_N_REL = 1000
_D = 64

_BR = 4000


def _tc_add_body(a_ref, b_ref, o_ref):
    o_ref[...] = a_ref[...] + b_ref[...]


_tc_ent_add = pl.pallas_call(
    _tc_add_body,
    grid=(_N_ENT // _BR,),
    in_specs=[
        pl.BlockSpec((_BR, _D), lambda i: (i, 0)),
        pl.BlockSpec((_BR, _D), lambda i: (i, 0)),
    ],
    out_specs=pl.BlockSpec((_BR, _D), lambda i: (i, 0)),
    out_shape=jax.ShapeDtypeStruct((_N_ENT, _D), jnp.float32),
)

_NUM_CORES = 2
_NUM_SUBCORES = 16
_LANES = 16
_RCH = 32
_REL_LAST = _N_REL - _RCH


def _add_rows(a, b, rows, rpi):
    def step(i, _):
        r0 = i * rpi
        for r in range(rpi):
            for j in range(_D // _LANES):
                sl = pl.ds(j * _LANES, _LANES)
                a[r0 + r, sl] = a[r0 + r, sl] + b[r0 + r, sl]
        return 0

    lax.fori_loop(0, rows // rpi, step, 0)


def _sc_rel_body(r0, r1, out_r, a, b):
    wid = lax.axis_index("s") * _NUM_CORES + lax.axis_index("c")
    base = pl.multiple_of(jnp.minimum(wid * _RCH, _REL_LAST), 8)
    rows = pl.ds(base, _RCH)
    pltpu.sync_copy(r0.at[rows], a)
    pltpu.sync_copy(r1.at[rows], b)
    _add_rows(a, b, _RCH, 8)
    pltpu.sync_copy(a, out_r.at[rows])


_sc_rel_add = pl.kernel(
    _sc_rel_body,
    out_type=jax.ShapeDtypeStruct((_N_REL, _D), jnp.float32),
    mesh=plsc.VectorSubcoreMesh(
        core_axis_name="c",
        subcore_axis_name="s",
        num_cores=_NUM_CORES,
        num_subcores=_NUM_SUBCORES,
    ),
    scratch_types=[
        pltpu.VMEM((_RCH, _D), jnp.float32),
        pltpu.VMEM((_RCH, _D), jnp.float32),
    ],
)


def kernel(inputs, ent_embeds_0, rel_embeds_0, ent_embeds_1, rel_embeds_1):
    del inputs
    out_ent = _tc_ent_add(ent_embeds_0, ent_embeds_1)
    out_rel = _sc_rel_add(rel_embeds_0, rel_embeds_1)
    return (out_ent, out_rel)


# speedup vs baseline: 1.3734x; 1.0834x over previous
import jax
import jax.numpy as jnp
from jax import lax
from jax.experimental import pallas as pl
from jax.experimental.pallas import tpu as pltpu
from jax.experimental.pallas import tpu_sc as plsc

_N_ENT = 100000
_N_REL = 1000
_D = 64

_BR = 4000


def _tc_add_body(a_ref, b_ref, o_ref):
    o_ref[...] = a_ref[...] + b_ref[...]


_tc_ent_add = pl.pallas_call(
    _tc_add_body,
    grid=(_N_ENT // _BR,),
    in_specs=[
        pl.BlockSpec((_BR, _D), lambda i: (i, 0)),
        pl.BlockSpec((_BR, _D), lambda i: (i, 0)),
    ],
    out_specs=pl.BlockSpec((_BR, _D), lambda i: (i, 0)),
    out_shape=jax.ShapeDtypeStruct((_N_ENT, _D), jnp.float32),
)

_tc_rel_add = pl.pallas_call(
    _tc_add_body,
    out_shape=jax.ShapeDtypeStruct((_N_REL, _D), jnp.float32),
)

_NUM_CORES = 2
_NUM_SUBCORES = 16
_LANES = 16
_RCH = 32
_REL_LAST = _N_REL - _RCH


def _add_rows(a, b, rows, rpi):
    def step(i, _):
        r0 = i * rpi
        for r in range(rpi):
            for j in range(_D // _LANES):
                sl = pl.ds(j * _LANES, _LANES)
                a[r0 + r, sl] = a[r0 + r, sl] + b[r0 + r, sl]
        return 0

    lax.fori_loop(0, rows // rpi, step, 0)


def _sc_rel_body(r0, r1, out_r, a, b):
    wid = lax.axis_index("s") * _NUM_CORES + lax.axis_index("c")
    base = pl.multiple_of(jnp.minimum(wid * _RCH, _REL_LAST), 8)
    rows = pl.ds(base, _RCH)
    pltpu.sync_copy(r0.at[rows], a)
    pltpu.sync_copy(r1.at[rows], b)
    _add_rows(a, b, _RCH, 8)
    pltpu.sync_copy(a, out_r.at[rows])


_sc_rel_add = pl.kernel(
    _sc_rel_body,
    out_type=jax.ShapeDtypeStruct((_N_REL, _D), jnp.float32),
    mesh=plsc.VectorSubcoreMesh(
        core_axis_name="c",
        subcore_axis_name="s",
        num_cores=_NUM_CORES,
        num_subcores=_NUM_SUBCORES,
    ),
    scratch_types=[
        pltpu.VMEM((_RCH, _D), jnp.float32),
        pltpu.VMEM((_RCH, _D), jnp.float32),
    ],
)


def kernel(inputs, ent_embeds_0, rel_embeds_0, ent_embeds_1, rel_embeds_1):
    del inputs
    out_ent = _tc_ent_add(ent_embeds_0, ent_embeds_1)
    out_rel = _tc_rel_add(rel_embeds_0, rel_embeds_1)
    return (out_ent, out_rel)
